# baseline (speedup 1.0000x reference)
import jax
import jax.numpy as jnp
from jax.experimental import pallas as pl
from jax.experimental.pallas import tpu as pltpu

_N = 1
_GH = 32
_GW = 32
_S = _GH * _GW
_D = 256
_NHEAD = 8
_DIM = _D // _NHEAD
_AW = 5
_C = max(_GH, _GW) * _AW
_SCALE = 16
_BQ = 1024
_NBLK = _S // _BQ

_NT = (((1,), (1,)), ((), ()))


def _perm_matrix():
    row = jax.lax.broadcasted_iota(jnp.int32, (_D, _D), 0)
    col = jax.lax.broadcasted_iota(jnp.int32, (_D, _D), 1)
    tgt = (col % _DIM) * _NHEAD + col // _DIM
    return (row == tgt).astype(jnp.float32)


def _bf16(a):
    return a.astype(jnp.bfloat16).astype(jnp.float32)


def _mm33(a, b):
    ab = [[_bf16(a[i][k]) for k in range(3)] for i in range(3)]
    bb = [[_bf16(b[k][j]) for j in range(3)] for k in range(3)]
    return [[ab[i][0] * bb[0][j] + ab[i][1] * bb[1][j] + ab[i][2] * bb[2][j]
             for j in range(3)] for i in range(3)]


def _attn_kernel(ik0_ref, ik1_ref, r_ref, t_ref, x_ref, s_ref, wq_ref, wk_ref,
                 wv_ref, wm_ref, w1_ref, w2_ref, g1_ref, b1_ref, g2_ref,
                 b2_ref, o_ref, k_s, v_s):
    f32 = jnp.float32
    bf16 = jnp.bfloat16
    p = _perm_matrix().astype(bf16)
    row = jax.lax.broadcasted_iota(jnp.int32, (_D, _D), 0)
    col = jax.lax.broadcasted_iota(jnp.int32, (_D, _D), 1)
    pt = (col == (row % _DIM) * _NHEAD + row // _DIM).astype(bf16)
    scale = jnp.float32(1.0 / (_DIM ** 0.5))
    wq_p = jnp.dot(pt, wq_ref[...].astype(bf16),
                   preferred_element_type=f32) * scale
    wm_p = jnp.dot(wm_ref[...].astype(bf16), p, preferred_element_type=f32)

    @pl.when(pl.program_id(0) == 0)
    def _project_kv():
        s = s_ref[...].astype(bf16)
        wk_p = jnp.dot(pt, wk_ref[...].astype(bf16), preferred_element_type=f32)
        wv_p = jnp.dot(pt, wv_ref[...].astype(bf16), preferred_element_type=f32)
        k_s[...] = jax.lax.dot_general(s, wk_p.astype(bf16), _NT,
                                       preferred_element_type=f32)
        v_s[...] = jax.lax.dot_general(s, wv_p.astype(bf16), _NT,
                                       preferred_element_type=f32)

    ik0 = ik0_ref[...]
    ik1 = ik1_ref[...]
    rm = r_ref[...]
    tv = t_ref[...]
    t0, t1, t2 = tv[0, 0:1], tv[0, 1:2], tv[0, 2:3]
    zero = jnp.zeros((1, 1), f32)
    a = [[ik1[0, 3 * j + i:3 * j + i + 1] for j in range(3)] for i in range(3)]
    sk = [[zero, -t2, t1], [t2, zero, -t0], [-t1, t0, zero]]
    rmat = [[rm[0, 3 * i + j:3 * i + j + 1] for j in range(3)] for i in range(3)]
    k0m = [[ik0[0, 3 * i + j:3 * i + j + 1] for j in range(3)] for i in range(3)]
    fm = _mm33(_mm33(_mm33(a, sk), rmat), k0m)

    rows = jax.lax.broadcasted_iota(jnp.int32, (_BQ, 1), 0)
    qx = (rows % _GW).astype(f32)
    qy = (rows // _GW).astype(f32)
    fb = [[_bf16(fm[i][j]) for j in range(3)] for i in range(3)]
    l0 = qx * fb[0][0] + qy * fb[0][1] + fb[0][2]
    l1 = qx * fb[1][0] + qy * fb[1][1] + fb[1][2]
    l2 = qx * fb[2][0] + qy * fb[2][1] + fb[2][2]

    cols = jax.lax.broadcasted_iota(jnp.int32, (1, _S), 1)
    sx = (cols % _GW).astype(jnp.float32)
    sy = (cols // _GW).astype(jnp.float32)
    half = jnp.float32(_AW // 2)
    r1 = jnp.float32(1.0) / l1
    r0 = jnp.float32(1.0) / l0
    j32 = jax.lax.broadcasted_iota(jnp.int32, (1, _GW), 1).astype(jnp.float32)
    gy = (l0 * j32 + l2) * r1
    gx = (l1 * j32 + l2) * r0
    gy_b = jnp.broadcast_to(gy.reshape(_BQ, 1, _GW),
                            (_BQ, _GH, _GW)).reshape(_BQ, _S)
    gx_b = jnp.broadcast_to(gx.reshape(_BQ, _GH, 1),
                            (_BQ, _GH, _GW)).reshape(_BQ, _S)
    dy = sy + gy_b
    dx = sx + gx_b
    wy = jnp.abs(dy) < half
    wx = jnp.abs(dx) < half
    mode = jnp.abs(l1) > jnp.abs(l0)
    within = (mode & wy) | (jnp.logical_not(mode) & wx)
    cnt = jnp.sum(within.astype(jnp.int32), axis=1, keepdims=True)
    novalid = cnt == 0
    mbias = jnp.where(within, jnp.float32(0.0), jnp.float32(-1e9))

    xb = x_ref[...]
    xbf = xb.astype(bf16)
    q = jax.lax.dot_general(xbf, wq_p.astype(bf16), _NT,
                            preferred_element_type=f32)
    qb = q.astype(bf16)
    k = k_s[...].astype(bf16)
    v = v_s[...].astype(bf16)
    ones_col = jnp.ones((_S, 1), bf16)
    v_fb = jnp.sum(v[:_C, :].astype(f32), axis=0, keepdims=True) \
        * jnp.float32(1.0 / _C)
    msg_parts = []
    for h in range(_NHEAD):
        qh = qb[:, h * _DIM:(h + 1) * _DIM]
        kh = k[:, h * _DIM:(h + 1) * _DIM]
        vh = v[:, h * _DIM:(h + 1) * _DIM]
        lg = jax.lax.dot_general(qh, kh, _NT, preferred_element_type=f32)
        e = jnp.exp(lg + mbias)
        vx = jnp.concatenate([vh, ones_col], axis=1)
        mhd = jnp.dot(e.astype(bf16), vx, preferred_element_type=f32)
        denom = mhd[:, _DIM:_DIM + 1]
        denom = jnp.where(novalid, jnp.float32(1.0), denom)
        mh = mhd[:, :_DIM] / denom
        mh = jnp.where(novalid, v_fb[:, h * _DIM:(h + 1) * _DIM], mh)
        msg_parts.append(mh)
    msg = jnp.concatenate(msg_parts, axis=1)

    merged = jax.lax.dot_general(msg.astype(bf16), wm_p.astype(bf16), _NT,
                                 preferred_element_type=f32)
    mu = jnp.mean(merged, axis=1, keepdims=True)
    var = jnp.mean((merged - mu) ** 2, axis=1, keepdims=True)
    msgn = (merged - mu) / jnp.sqrt(var + 1e-5) * g1_ref[...] + b1_ref[...]

    w1 = w1_ref[...].astype(bf16)
    h1 = (jax.lax.dot_general(xbf, w1[:, :_D], _NT, preferred_element_type=f32)
          + jax.lax.dot_general(msgn.astype(bf16), w1[:, _D:], _NT,
                                preferred_element_type=f32))
    h1 = jnp.maximum(h1, jnp.float32(0.0))
    h2 = jax.lax.dot_general(h1.astype(bf16), w2_ref[...].astype(bf16), _NT,
                             preferred_element_type=f32)
    mu2 = jnp.mean(h2, axis=1, keepdims=True)
    var2 = jnp.mean((h2 - mu2) ** 2, axis=1, keepdims=True)
    h2n = (h2 - mu2) / jnp.sqrt(var2 + 1e-5) * g2_ref[...] + b2_ref[...]
    o_ref[...] = xb + h2n


def kernel(x, source, K0, K1, R, t, Wq, Wk, Wv, Wmerge, Wmlp1, Wmlp2,
           ln1_g, ln1_b, ln2_g, ln2_b):
    K0s = jnp.concatenate([K0[:, :2, :] / _SCALE, K0[:, 2:, :]], axis=1)
    K1s = jnp.concatenate([K1[:, :2, :] / _SCALE, K1[:, 2:, :]], axis=1)
    ik0 = jnp.linalg.inv(K0s).reshape(1, 9)
    ik1 = jnp.linalg.inv(K1s).reshape(1, 9)
    rflat = R.reshape(1, 9)
    tflat = t.reshape(1, 3)

    g1 = ln1_g.reshape(1, _D)
    b1 = ln1_b.reshape(1, _D)
    g2 = ln2_g.reshape(1, _D)
    b2 = ln2_b.reshape(1, _D)
    x2 = x.reshape(_S, _D)
    s2 = source.reshape(_S, _D)

    out = pl.pallas_call(
        _attn_kernel,
        grid=(_NBLK,),
        in_specs=[
            pl.BlockSpec((1, 9), lambda i: (0, 0)),
            pl.BlockSpec((1, 9), lambda i: (0, 0)),
            pl.BlockSpec((1, 9), lambda i: (0, 0)),
            pl.BlockSpec((1, 3), lambda i: (0, 0)),
            pl.BlockSpec((_BQ, _D), lambda i: (i, 0)),
            pl.BlockSpec((_S, _D), lambda i: (0, 0)),
            pl.BlockSpec((_D, _D), lambda i: (0, 0)),
            pl.BlockSpec((_D, _D), lambda i: (0, 0)),
            pl.BlockSpec((_D, _D), lambda i: (0, 0)),
            pl.BlockSpec((_D, _D), lambda i: (0, 0)),
            pl.BlockSpec((2 * _D, 2 * _D), lambda i: (0, 0)),
            pl.BlockSpec((_D, 2 * _D), lambda i: (0, 0)),
            pl.BlockSpec((1, _D), lambda i: (0, 0)),
            pl.BlockSpec((1, _D), lambda i: (0, 0)),
            pl.BlockSpec((1, _D), lambda i: (0, 0)),
            pl.BlockSpec((1, _D), lambda i: (0, 0)),
        ],
        out_specs=pl.BlockSpec((_BQ, _D), lambda i: (i, 0)),
        out_shape=jax.ShapeDtypeStruct((_S, _D), jnp.float32),
        scratch_shapes=[
            pltpu.VMEM((_S, _D), jnp.float32),
            pltpu.VMEM((_S, _D), jnp.float32),
        ],
    )(ik0, ik1, rflat, tflat, x2, s2, Wq, Wk, Wv, Wmerge, Wmlp1, Wmlp2,
      g1, b1, g2, b2)

    return out.reshape(_N, _S, _D)


# speedup vs baseline: 1.2171x; 1.2171x over previous
import jax
import jax.numpy as jnp
from jax.experimental import pallas as pl
from jax.experimental.pallas import tpu as pltpu

_N = 1
_GH = 32
_GW = 32
_S = _GH * _GW
_D = 256
_NHEAD = 8
_DIM = _D // _NHEAD
_AW = 5
_C = max(_GH, _GW) * _AW
_SCALE = 16
_BQ = 512
_NBLK = _S // _BQ

_NT = (((1,), (1,)), ((), ()))


def _perm_matrix():
    row = jax.lax.broadcasted_iota(jnp.int32, (_D, _D), 0)
    col = jax.lax.broadcasted_iota(jnp.int32, (_D, _D), 1)
    tgt = (col % _DIM) * _NHEAD + col // _DIM
    return (row == tgt).astype(jnp.float32)


def _bf16(a):
    return a.astype(jnp.bfloat16).astype(jnp.float32)


def _mm33(a, b):
    ab = [[_bf16(a[i][k]) for k in range(3)] for i in range(3)]
    bb = [[_bf16(b[k][j]) for j in range(3)] for k in range(3)]
    return [[ab[i][0] * bb[0][j] + ab[i][1] * bb[1][j] + ab[i][2] * bb[2][j]
             for j in range(3)] for i in range(3)]


def _attn_kernel(ik0_ref, ik1_ref, r_ref, t_ref, x_ref, s_ref, wq_ref, wk_ref,
                 wv_ref, wm_ref, w1_ref, w2_ref, g1_ref, b1_ref, g2_ref,
                 b2_ref, o_ref):
    f32 = jnp.float32
    bf16 = jnp.bfloat16
    p = _perm_matrix().astype(bf16)
    row = jax.lax.broadcasted_iota(jnp.int32, (_D, _D), 0)
    col = jax.lax.broadcasted_iota(jnp.int32, (_D, _D), 1)
    pt = (col == (row % _DIM) * _NHEAD + row // _DIM).astype(bf16)
    scale = jnp.float32(1.0 / (_DIM ** 0.5))
    wq_p = jnp.dot(pt, wq_ref[...].astype(bf16),
                   preferred_element_type=f32) * scale
    wm_p = jnp.dot(wm_ref[...].astype(bf16), p, preferred_element_type=f32)

    s = s_ref[...].astype(bf16)
    wk_p = jnp.dot(pt, wk_ref[...].astype(bf16), preferred_element_type=f32)
    wv_p = jnp.dot(pt, wv_ref[...].astype(bf16), preferred_element_type=f32)
    k_full = jax.lax.dot_general(s, wk_p.astype(bf16), _NT,
                                 preferred_element_type=f32)
    v_full = jax.lax.dot_general(s, wv_p.astype(bf16), _NT,
                                 preferred_element_type=f32)

    ik0 = ik0_ref[...]
    ik1 = ik1_ref[...]
    rm = r_ref[...]
    tv = t_ref[...]
    t0, t1, t2 = tv[0, 0:1], tv[0, 1:2], tv[0, 2:3]
    zero = jnp.zeros((1, 1), f32)
    a = [[ik1[0, 3 * j + i:3 * j + i + 1] for j in range(3)] for i in range(3)]
    sk = [[zero, -t2, t1], [t2, zero, -t0], [-t1, t0, zero]]
    rmat = [[rm[0, 3 * i + j:3 * i + j + 1] for j in range(3)] for i in range(3)]
    k0m = [[ik0[0, 3 * i + j:3 * i + j + 1] for j in range(3)] for i in range(3)]
    fm = _mm33(_mm33(_mm33(a, sk), rmat), k0m)

    rows = jax.lax.broadcasted_iota(jnp.int32, (_BQ, 1), 0) \
        + pl.program_id(0) * _BQ
    qx = (rows % _GW).astype(f32)
    qy = (rows // _GW).astype(f32)
    fb = [[_bf16(fm[i][j]) for j in range(3)] for i in range(3)]
    l0 = qx * fb[0][0] + qy * fb[0][1] + fb[0][2]
    l1 = qx * fb[1][0] + qy * fb[1][1] + fb[1][2]
    l2 = qx * fb[2][0] + qy * fb[2][1] + fb[2][2]

    cols = jax.lax.broadcasted_iota(jnp.int32, (1, _S), 1)
    sx = (cols % _GW).astype(jnp.float32)
    sy = (cols // _GW).astype(jnp.float32)
    half = jnp.float32(_AW // 2)
    r1 = jnp.float32(1.0) / l1
    r0 = jnp.float32(1.0) / l0
    dy = sy + (l0 * sx + l2) * r1
    dx = sx + (l1 * sy + l2) * r0
    wy = jnp.abs(dy) < half
    wx = jnp.abs(dx) < half
    mode = jnp.abs(l1) > jnp.abs(l0)
    within = (mode & wy) | (jnp.logical_not(mode) & wx)
    cnt = jnp.sum(within.astype(jnp.int32), axis=1, keepdims=True)
    novalid = cnt == 0
    mbias = jnp.where(within, jnp.float32(0.0), jnp.float32(-1e9))

    xb = x_ref[...]
    xbf = xb.astype(bf16)
    q = jax.lax.dot_general(xbf, wq_p.astype(bf16), _NT,
                            preferred_element_type=f32)
    qb = q.astype(bf16)
    k = k_full.astype(bf16)
    v = v_full.astype(bf16)
    ones_col = jnp.ones((_S, 1), bf16)
    v_fb = jnp.sum(v[:_C, :].astype(f32), axis=0, keepdims=True) \
        * jnp.float32(1.0 / _C)
    msg_parts = []
    for h in range(_NHEAD):
        qh = qb[:, h * _DIM:(h + 1) * _DIM]
        kh = k[:, h * _DIM:(h + 1) * _DIM]
        vh = v[:, h * _DIM:(h + 1) * _DIM]
        lg = jax.lax.dot_general(qh, kh, _NT, preferred_element_type=f32)
        e = jnp.exp(lg + mbias)
        vx = jnp.concatenate([vh, ones_col], axis=1)
        mhd = jnp.dot(e.astype(bf16), vx, preferred_element_type=f32)
        denom = mhd[:, _DIM:_DIM + 1]
        denom = jnp.where(novalid, jnp.float32(1.0), denom)
        mh = mhd[:, :_DIM] / denom
        mh = jnp.where(novalid, v_fb[:, h * _DIM:(h + 1) * _DIM], mh)
        msg_parts.append(mh)
    msg = jnp.concatenate(msg_parts, axis=1)

    merged = jax.lax.dot_general(msg.astype(bf16), wm_p.astype(bf16), _NT,
                                 preferred_element_type=f32)
    mu = jnp.mean(merged, axis=1, keepdims=True)
    var = jnp.mean((merged - mu) ** 2, axis=1, keepdims=True)
    msgn = (merged - mu) / jnp.sqrt(var + 1e-5) * g1_ref[...] + b1_ref[...]

    w1 = w1_ref[...].astype(bf16)
    h1 = (jax.lax.dot_general(xbf, w1[:, :_D], _NT, preferred_element_type=f32)
          + jax.lax.dot_general(msgn.astype(bf16), w1[:, _D:], _NT,
                                preferred_element_type=f32))
    h1 = jnp.maximum(h1, jnp.float32(0.0))
    h2 = jax.lax.dot_general(h1.astype(bf16), w2_ref[...].astype(bf16), _NT,
                             preferred_element_type=f32)
    mu2 = jnp.mean(h2, axis=1, keepdims=True)
    var2 = jnp.mean((h2 - mu2) ** 2, axis=1, keepdims=True)
    h2n = (h2 - mu2) / jnp.sqrt(var2 + 1e-5) * g2_ref[...] + b2_ref[...]
    o_ref[...] = xb + h2n


def kernel(x, source, K0, K1, R, t, Wq, Wk, Wv, Wmerge, Wmlp1, Wmlp2,
           ln1_g, ln1_b, ln2_g, ln2_b):
    K0s = jnp.concatenate([K0[:, :2, :] / _SCALE, K0[:, 2:, :]], axis=1)
    K1s = jnp.concatenate([K1[:, :2, :] / _SCALE, K1[:, 2:, :]], axis=1)
    ik0 = jnp.linalg.inv(K0s).reshape(1, 9)
    ik1 = jnp.linalg.inv(K1s).reshape(1, 9)
    rflat = R.reshape(1, 9)
    tflat = t.reshape(1, 3)

    g1 = ln1_g.reshape(1, _D)
    b1 = ln1_b.reshape(1, _D)
    g2 = ln2_g.reshape(1, _D)
    b2 = ln2_b.reshape(1, _D)
    x2 = x.reshape(_S, _D)
    s2 = source.reshape(_S, _D)

    out = pl.pallas_call(
        _attn_kernel,
        grid=(_NBLK,),
        in_specs=[
            pl.BlockSpec((1, 9), lambda i: (0, 0)),
            pl.BlockSpec((1, 9), lambda i: (0, 0)),
            pl.BlockSpec((1, 9), lambda i: (0, 0)),
            pl.BlockSpec((1, 3), lambda i: (0, 0)),
            pl.BlockSpec((_BQ, _D), lambda i: (i, 0)),
            pl.BlockSpec((_S, _D), lambda i: (0, 0)),
            pl.BlockSpec((_D, _D), lambda i: (0, 0)),
            pl.BlockSpec((_D, _D), lambda i: (0, 0)),
            pl.BlockSpec((_D, _D), lambda i: (0, 0)),
            pl.BlockSpec((_D, _D), lambda i: (0, 0)),
            pl.BlockSpec((2 * _D, 2 * _D), lambda i: (0, 0)),
            pl.BlockSpec((_D, 2 * _D), lambda i: (0, 0)),
            pl.BlockSpec((1, _D), lambda i: (0, 0)),
            pl.BlockSpec((1, _D), lambda i: (0, 0)),
            pl.BlockSpec((1, _D), lambda i: (0, 0)),
            pl.BlockSpec((1, _D), lambda i: (0, 0)),
        ],
        out_specs=pl.BlockSpec((_BQ, _D), lambda i: (i, 0)),
        out_shape=jax.ShapeDtypeStruct((_S, _D), jnp.float32),
        compiler_params=pltpu.CompilerParams(
            dimension_semantics=("parallel",)),
    )(ik0, ik1, rflat, tflat, x2, s2, Wq, Wk, Wv, Wmerge, Wmlp1, Wmlp2,
      g1, b1, g2, b2)

    return out.reshape(_N, _S, _D)


# speedup vs baseline: 1.2603x; 1.0355x over previous
import jax
import jax.numpy as jnp
from jax.experimental import pallas as pl
from jax.experimental.pallas import tpu as pltpu

_N = 1
_GH = 32
_GW = 32
_S = _GH * _GW
_D = 256
_NHEAD = 8
_DIM = _D // _NHEAD
_AW = 5
_C = max(_GH, _GW) * _AW
_SCALE = 16
_BQ = 1024
_NBLK = _S // _BQ

_NT = (((1,), (1,)), ((), ()))


def _perm_matrix():
    row = jax.lax.broadcasted_iota(jnp.int32, (_D, _D), 0)
    col = jax.lax.broadcasted_iota(jnp.int32, (_D, _D), 1)
    tgt = (col % _DIM) * _NHEAD + col // _DIM
    return (row == tgt).astype(jnp.float32)


def _bf16(a):
    return a.astype(jnp.bfloat16).astype(jnp.float32)


def _mm33(a, b):
    ab = [[_bf16(a[i][k]) for k in range(3)] for i in range(3)]
    bb = [[_bf16(b[k][j]) for j in range(3)] for k in range(3)]
    return [[ab[i][0] * bb[0][j] + ab[i][1] * bb[1][j] + ab[i][2] * bb[2][j]
             for j in range(3)] for i in range(3)]


def _attn_kernel(ik0_ref, ik1_ref, r_ref, t_ref, x_ref, s_ref, wq_ref, wk_ref,
                 wv_ref, wm_ref, w1_ref, w2_ref, g1_ref, b1_ref, g2_ref,
                 b2_ref, o_ref, k_s, v_s):
    f32 = jnp.float32
    bf16 = jnp.bfloat16
    p = _perm_matrix().astype(bf16)
    row = jax.lax.broadcasted_iota(jnp.int32, (_D, _D), 0)
    col = jax.lax.broadcasted_iota(jnp.int32, (_D, _D), 1)
    pt = (col == (row % _DIM) * _NHEAD + row // _DIM).astype(bf16)
    scale = jnp.float32(1.0 / (_DIM ** 0.5))
    wq_p = jnp.dot(pt, wq_ref[...].astype(bf16),
                   preferred_element_type=f32) * scale
    wm_p = jnp.dot(wm_ref[...].astype(bf16), p, preferred_element_type=f32)

    @pl.when(pl.program_id(0) == 0)
    def _project_kv():
        s = s_ref[...].astype(bf16)
        wk_p = jnp.dot(pt, wk_ref[...].astype(bf16), preferred_element_type=f32)
        wv_p = jnp.dot(pt, wv_ref[...].astype(bf16), preferred_element_type=f32)
        k_s[...] = jax.lax.dot_general(s, wk_p.astype(bf16), _NT,
                                       preferred_element_type=f32)
        v_s[...] = jax.lax.dot_general(s, wv_p.astype(bf16), _NT,
                                       preferred_element_type=f32)

    ik0 = ik0_ref[...]
    ik1 = ik1_ref[...]
    rm = r_ref[...]
    tv = t_ref[...]
    t0, t1, t2 = tv[0, 0:1], tv[0, 1:2], tv[0, 2:3]
    zero = jnp.zeros((1, 1), f32)
    a = [[ik1[0, 3 * j + i:3 * j + i + 1] for j in range(3)] for i in range(3)]
    sk = [[zero, -t2, t1], [t2, zero, -t0], [-t1, t0, zero]]
    rmat = [[rm[0, 3 * i + j:3 * i + j + 1] for j in range(3)] for i in range(3)]
    k0m = [[ik0[0, 3 * i + j:3 * i + j + 1] for j in range(3)] for i in range(3)]
    fm = _mm33(_mm33(_mm33(a, sk), rmat), k0m)

    rows = jax.lax.broadcasted_iota(jnp.int32, (_BQ, 1), 0)
    qx = (rows % _GW).astype(f32)
    qy = (rows // _GW).astype(f32)
    fb = [[_bf16(fm[i][j]) for j in range(3)] for i in range(3)]
    l0 = qx * fb[0][0] + qy * fb[0][1] + fb[0][2]
    l1 = qx * fb[1][0] + qy * fb[1][1] + fb[1][2]
    l2 = qx * fb[2][0] + qy * fb[2][1] + fb[2][2]

    cols = jax.lax.broadcasted_iota(jnp.int32, (1, _S), 1)
    sx = (cols % _GW).astype(jnp.float32)
    sy = (cols // _GW).astype(jnp.float32)
    half = jnp.float32(_AW // 2)
    r1 = jnp.float32(1.0) / l1
    r0 = jnp.float32(1.0) / l0
    dy = sy + (l0 * sx + l2) * r1
    dx = sx + (l1 * sy + l2) * r0
    wy = jnp.abs(dy) < half
    wx = jnp.abs(dx) < half
    mode = jnp.abs(l1) > jnp.abs(l0)
    within = wx ^ (mode & (wy ^ wx))
    mbias = jnp.where(within, jnp.float32(0.0), jnp.float32(-1e9))
    novalid = jnp.max(mbias, axis=1, keepdims=True) < jnp.float32(-0.5)

    xb = x_ref[...]
    xbf = xb.astype(bf16)
    q = jax.lax.dot_general(xbf, wq_p.astype(bf16), _NT,
                            preferred_element_type=f32)
    qb = q.astype(bf16)
    k = k_s[...].astype(bf16)
    v = v_s[...].astype(bf16)
    ones_col = jnp.ones((_S, 1), bf16)
    v_fb = jnp.sum(v[:_C, :].astype(f32), axis=0, keepdims=True) \
        * jnp.float32(1.0 / _C)
    msg_parts = []
    for h in range(_NHEAD):
        qh = qb[:, h * _DIM:(h + 1) * _DIM]
        kh = k[:, h * _DIM:(h + 1) * _DIM]
        vh = v[:, h * _DIM:(h + 1) * _DIM]
        lg = jax.lax.dot_general(qh, kh, _NT, preferred_element_type=f32)
        e = jnp.exp(lg + mbias)
        vx = jnp.concatenate([vh, ones_col], axis=1)
        mhd = jnp.dot(e.astype(bf16), vx, preferred_element_type=f32)
        denom = mhd[:, _DIM:_DIM + 1]
        denom = jnp.where(novalid, jnp.float32(1.0), denom)
        mh = mhd[:, :_DIM] * (jnp.float32(1.0) / denom)
        mh = jnp.where(novalid, v_fb[:, h * _DIM:(h + 1) * _DIM], mh)
        msg_parts.append(mh)
    msg = jnp.concatenate(msg_parts, axis=1)

    merged = jax.lax.dot_general(msg.astype(bf16), wm_p.astype(bf16), _NT,
                                 preferred_element_type=f32)
    mu = jnp.mean(merged, axis=1, keepdims=True)
    var = jnp.mean((merged - mu) ** 2, axis=1, keepdims=True)
    msgn = (merged - mu) / jnp.sqrt(var + 1e-5) * g1_ref[...] + b1_ref[...]

    w1 = w1_ref[...].astype(bf16)
    h1 = (jax.lax.dot_general(xbf, w1[:, :_D], _NT, preferred_element_type=f32)
          + jax.lax.dot_general(msgn.astype(bf16), w1[:, _D:], _NT,
                                preferred_element_type=f32))
    h1 = jnp.maximum(h1, jnp.float32(0.0))
    h2 = jax.lax.dot_general(h1.astype(bf16), w2_ref[...].astype(bf16), _NT,
                             preferred_element_type=f32)
    mu2 = jnp.mean(h2, axis=1, keepdims=True)
    var2 = jnp.mean((h2 - mu2) ** 2, axis=1, keepdims=True)
    h2n = (h2 - mu2) / jnp.sqrt(var2 + 1e-5) * g2_ref[...] + b2_ref[...]
    o_ref[...] = xb + h2n


def kernel(x, source, K0, K1, R, t, Wq, Wk, Wv, Wmerge, Wmlp1, Wmlp2,
           ln1_g, ln1_b, ln2_g, ln2_b):
    K0s = jnp.concatenate([K0[:, :2, :] / _SCALE, K0[:, 2:, :]], axis=1)
    K1s = jnp.concatenate([K1[:, :2, :] / _SCALE, K1[:, 2:, :]], axis=1)
    ik0 = jnp.linalg.inv(K0s).reshape(1, 9)
    ik1 = jnp.linalg.inv(K1s).reshape(1, 9)
    rflat = R.reshape(1, 9)
    tflat = t.reshape(1, 3)

    g1 = ln1_g.reshape(1, _D)
    b1 = ln1_b.reshape(1, _D)
    g2 = ln2_g.reshape(1, _D)
    b2 = ln2_b.reshape(1, _D)
    x2 = x.reshape(_S, _D)
    s2 = source.reshape(_S, _D)

    out = pl.pallas_call(
        _attn_kernel,
        grid=(_NBLK,),
        in_specs=[
            pl.BlockSpec((1, 9), lambda i: (0, 0)),
            pl.BlockSpec((1, 9), lambda i: (0, 0)),
            pl.BlockSpec((1, 9), lambda i: (0, 0)),
            pl.BlockSpec((1, 3), lambda i: (0, 0)),
            pl.BlockSpec((_BQ, _D), lambda i: (i, 0)),
            pl.BlockSpec((_S, _D), lambda i: (0, 0)),
            pl.BlockSpec((_D, _D), lambda i: (0, 0)),
            pl.BlockSpec((_D, _D), lambda i: (0, 0)),
            pl.BlockSpec((_D, _D), lambda i: (0, 0)),
            pl.BlockSpec((_D, _D), lambda i: (0, 0)),
            pl.BlockSpec((2 * _D, 2 * _D), lambda i: (0, 0)),
            pl.BlockSpec((_D, 2 * _D), lambda i: (0, 0)),
            pl.BlockSpec((1, _D), lambda i: (0, 0)),
            pl.BlockSpec((1, _D), lambda i: (0, 0)),
            pl.BlockSpec((1, _D), lambda i: (0, 0)),
            pl.BlockSpec((1, _D), lambda i: (0, 0)),
        ],
        out_specs=pl.BlockSpec((_BQ, _D), lambda i: (i, 0)),
        out_shape=jax.ShapeDtypeStruct((_S, _D), jnp.float32),
        scratch_shapes=[
            pltpu.VMEM((_S, _D), jnp.float32),
            pltpu.VMEM((_S, _D), jnp.float32),
        ],
    )(ik0, ik1, rflat, tflat, x2, s2, Wq, Wk, Wv, Wmerge, Wmlp1, Wmlp2,
      g1, b1, g2, b2)

    return out.reshape(_N, _S, _D)
